# Initial kernel scaffold; baseline (speedup 1.0000x reference)
import jax
import jax.numpy as jnp
from jax.experimental import pallas as pl


def kernel(hidden_states, Wq, bq, Wk, bk, Wv, bv, Wl, bl):
    raise NotImplementedError("write your pallas kernel here")


# speedup vs baseline: 4.0649x; 4.0649x over previous
import functools
import math

import jax
import jax.numpy as jnp
import numpy as np
from jax.experimental import pallas as pl
from jax.experimental.pallas import tpu as pltpu


F32 = jnp.float32


@functools.lru_cache(maxsize=None)
def _dft_consts(S: int, FP: int, H: int):
    F = S // 2 + 1
    t = np.arange(S, dtype=np.float64)[:, None]
    f = np.arange(FP, dtype=np.float64)[None, :]
    ang = (2.0 * np.pi / S) * t * f
    cr = np.cos(ang)
    ci = -np.sin(ang)
    cr[:, F:] = 0.0
    ci[:, F:] = 0.0
    cosm = cr.astype(np.float32)
    sinm = ci.astype(np.float32)
    wf = np.full((FP, 1), 2.0, dtype=np.float64)
    wf[0, 0] = 1.0
    wf[S // 2, 0] = 1.0
    wf[F:, 0] = 0.0
    tau = np.arange(S, dtype=np.float64)[None, :]
    fa = np.arange(FP, dtype=np.float64)[:, None]
    ang2 = (2.0 * np.pi / S) * fa * tau
    scale = wf / (S * H)
    ar = (np.cos(ang2) * scale).astype(np.float32)
    ai = (-np.sin(ang2) * scale).astype(np.float32)
    return cosm, sinm, ar, ai


def _qk_body(xt_ref, wq_ref, wk_ref, bq_ref, bk_ref, qt_ref, kt_ref):
    xt = xt_ref[0]
    qt_ref[0] = jnp.dot(wq_ref[0], xt, preferred_element_type=F32) + bq_ref[0]
    kt_ref[0] = jnp.dot(wk_ref[0], xt, preferred_element_type=F32) + bk_ref[0]


def _spec_body(qt_ref, kt_ref, cos_ref, sin_ref, p_ref, acc_ref):
    h = pl.program_id(2)
    hp = jax.lax.Precision.HIGHEST
    qt = qt_ref[0]
    kt = kt_ref[0]
    qr = jnp.dot(qt, cos_ref[...], preferred_element_type=F32, precision=hp)
    qi = jnp.dot(qt, sin_ref[...], preferred_element_type=F32, precision=hp)
    kr = jnp.dot(kt, cos_ref[...], preferred_element_type=F32, precision=hp)
    ki = jnp.dot(kt, sin_ref[...], preferred_element_type=F32, precision=hp)
    pre = qr * kr + qi * ki
    pim = qi * kr - qr * ki

    @pl.when(h == 0)
    def _():
        acc_ref[...] = jnp.zeros_like(acc_ref)

    E = acc_ref.shape[0] // 2
    acc_ref[0:E, :] = acc_ref[0:E, :] + pre
    acc_ref[E:, :] = acc_ref[E:, :] + pim

    @pl.when(h == pl.num_programs(2) - 1)
    def _():
        p_ref[0] = acc_ref[...]


def _irfft_body(p_ref, ar_ref, ai_ref, m_ref):
    E = p_ref.shape[1] // 2
    pr = p_ref[0, 0:E, :]
    pi = p_ref[0, E:, :]
    m_ref[0] = (jnp.dot(pr, ar_ref[...], preferred_element_type=F32, precision=jax.lax.Precision.HIGHEST)
                + jnp.dot(pi, ai_ref[...], preferred_element_type=F32, precision=jax.lax.Precision.HIGHEST))


def _values_body(x_ref, wv_ref, bv_ref, v_ref):
    v_ref[0] = jnp.dot(x_ref[0], wv_ref[...],
                       preferred_element_type=F32) + bv_ref[0]


def _topk_body(m_ref, a_ref, ap_ref, w_ref, mwork_ref, *, K, S):
    E = m_ref.shape[1]
    mwork_ref[...] = m_ref[0]
    lane = jax.lax.broadcasted_iota(jnp.int32, (E, S), 1)
    vlane = jax.lax.broadcasted_iota(jnp.int32, (E, 128), 1)

    def body(i, carry):
        vals, _ = carry
        cur = mwork_ref[...]
        mx = jnp.max(cur, axis=1, keepdims=True)
        am = jnp.min(jnp.where(cur == mx, lane, S), axis=1, keepdims=True)
        vals = jnp.where(vlane == i, mx, vals)
        mwork_ref[...] = jnp.where(lane == am, -jnp.inf, cur)
        return vals, am

    vals, d = jax.lax.fori_loop(
        0, K, body,
        (jnp.zeros((E, 128), F32), jnp.zeros((E, 1), jnp.int32)))
    ez = jnp.where(vlane < K, jnp.exp(vals - vals[:, 0:1]), 0.0)
    denom = jnp.sum(ez, axis=1, keepdims=True)
    w_ref[0] = ez[:, K - 1:K] / denom
    a_ref[0] = d >> 1
    ap_ref[0] = (d >> 1) + (d & 1)


def _gather_body(a_ref, ap_ref, v_ref, ge_ref, go_ref, *, M, HP, G, E):
    b = pl.program_id(0)
    et = pl.program_id(1)
    base = b * E + et * G
    grp = jax.lax.broadcasted_iota(jnp.int32, (M, G * HP), 1) // HP
    ge = None
    go = None
    for j in range(G):
        a = a_ref[base + j]
        ap = ap_ref[base + j]
        se = v_ref[0, pl.ds(a, M), :]
        so = v_ref[0, pl.ds(ap, M), :]
        ge = se if j == 0 else jnp.where(grp == j, se, ge)
        go = so if j == 0 else jnp.where(grp == j, so, go)
    ge_ref[0] = ge
    go_ref[0] = go


def _final_body(ge_ref, go_ref, w_ref, wl_ref, bl_ref, oe_ref, oo_ref):
    w = w_ref[0]
    wl = wl_ref[...]
    bl = bl_ref[0]
    oe_ref[0] = jnp.dot(ge_ref[0] * w, wl, preferred_element_type=F32, precision=jax.lax.Precision.HIGHEST) + bl
    oo_ref[0] = jnp.dot(go_ref[0] * w, wl, preferred_element_type=F32, precision=jax.lax.Precision.HIGHEST) + bl


def kernel(hidden_states, Wq, bq, Wk, bk, Wv, bv, Wl, bl):
    B, S, D = hidden_states.shape
    _, E, H = Wq.shape
    F = S // 2 + 1
    FP = F + 127
    HP = 16
    EH = E * HP
    K = int(3 * math.log(float(S)))
    M = S // 2
    cos_np, sin_np, ar_np, ai_np = _dft_consts(S, FP, H)

    x = hidden_states.astype(F32)
    xt = jnp.transpose(x, (0, 2, 1))
    cosm = jnp.asarray(cos_np)
    sinm = jnp.asarray(sin_np)
    ar = jnp.asarray(ar_np)
    ai = jnp.asarray(ai_np)
    wqT = jnp.transpose(Wq, (2, 1, 0))
    wkT = jnp.transpose(Wk, (2, 1, 0))
    bqT = jnp.transpose(bq, (1, 0))[:, :, None]
    bkT = jnp.transpose(bk, (1, 0))[:, :, None]
    wv2 = jnp.pad(Wv, ((0, 0), (0, 0), (0, HP - H))).reshape(D, EH)
    bv2 = jnp.pad(bv, ((0, 0), (0, HP - H))).reshape(1, EH)
    wl2 = jnp.pad(Wl, ((0, 0), (0, HP - H), (0, 0))).reshape(EH, D)
    bl3 = bl.reshape(1, D)

    qt, kt = pl.pallas_call(
        _qk_body,
        grid=(B, H),
        in_specs=[pl.BlockSpec((1, D, S), lambda b, h: (b, 0, 0)),
                  pl.BlockSpec((1, E, D), lambda b, h: (h, 0, 0)),
                  pl.BlockSpec((1, E, D), lambda b, h: (h, 0, 0)),
                  pl.BlockSpec((1, E, 1), lambda b, h: (h, 0, 0)),
                  pl.BlockSpec((1, E, 1), lambda b, h: (h, 0, 0))],
        out_specs=[pl.BlockSpec((1, E, S), lambda b, h, H=H: (b * H + h, 0, 0)),
                   pl.BlockSpec((1, E, S), lambda b, h, H=H: (b * H + h, 0, 0))],
        out_shape=[jax.ShapeDtypeStruct((B * H, E, S), F32),
                   jax.ShapeDtypeStruct((B * H, E, S), F32)],
    )(xt, wqT, wkT, bqT, bkT)

    FT = 3
    FPT = FP // FT
    p = pl.pallas_call(
        _spec_body,
        grid=(B, FT, H),
        in_specs=[pl.BlockSpec((1, E, S), lambda b, ft, h, H=H: (b * H + h, 0, 0)),
                  pl.BlockSpec((1, E, S), lambda b, ft, h, H=H: (b * H + h, 0, 0)),
                  pl.BlockSpec((S, FPT), lambda b, ft, h: (0, ft)),
                  pl.BlockSpec((S, FPT), lambda b, ft, h: (0, ft))],
        out_specs=pl.BlockSpec((1, 2 * E, FPT), lambda b, ft, h: (b, 0, ft)),
        out_shape=jax.ShapeDtypeStruct((B, 2 * E, FP), F32),
        scratch_shapes=[pltpu.VMEM((2 * E, FPT), F32)],
    )(qt, kt, cosm, sinm)

    ST3 = 2
    m = pl.pallas_call(
        _irfft_body,
        grid=(B, ST3),
        in_specs=[pl.BlockSpec((1, 2 * E, FP), lambda b, s: (b, 0, 0)),
                  pl.BlockSpec((FP, S // ST3), lambda b, s: (0, s)),
                  pl.BlockSpec((FP, S // ST3), lambda b, s: (0, s))],
        out_specs=pl.BlockSpec((1, E, S // ST3), lambda b, s: (b, 0, s)),
        out_shape=jax.ShapeDtypeStruct((B, E, S), F32),
    )(p, ar, ai)

    ST = 4
    v = pl.pallas_call(
        _values_body,
        grid=(B, ST),
        in_specs=[pl.BlockSpec((1, S // ST, D), lambda b, s: (b, s, 0)),
                  pl.BlockSpec((D, EH), lambda b, s: (0, 0)),
                  pl.BlockSpec((1, EH), lambda b, s: (0, 0))],
        out_specs=pl.BlockSpec((1, S // ST, EH), lambda b, s: (b, s, 0)),
        out_shape=jax.ShapeDtypeStruct((B, S, EH), F32),
    )(x, wv2, bv2)

    a_i, ap_i, w = pl.pallas_call(
        functools.partial(_topk_body, K=K, S=S),
        grid=(B,),
        in_specs=[pl.BlockSpec((1, E, S), lambda b: (b, 0, 0))],
        out_specs=[pl.BlockSpec((1, E, 1), lambda b: (b, 0, 0)),
                   pl.BlockSpec((1, E, 1), lambda b: (b, 0, 0)),
                   pl.BlockSpec((1, E, 1), lambda b: (b, 0, 0))],
        out_shape=[jax.ShapeDtypeStruct((B, E, 1), jnp.int32),
                   jax.ShapeDtypeStruct((B, E, 1), jnp.int32),
                   jax.ShapeDtypeStruct((B, E, 1), F32)],
        scratch_shapes=[pltpu.VMEM((E, S), F32)],
    )(m)

    a_flat = a_i.reshape(B * E)
    ap_flat = ap_i.reshape(B * E)
    w_row = jnp.repeat(w.reshape(B, E), HP, axis=1).reshape(B, 1, EH)

    G = 8
    ge, go = pl.pallas_call(
        functools.partial(_gather_body, M=M, HP=HP, G=G, E=E),
        grid_spec=pltpu.PrefetchScalarGridSpec(
            num_scalar_prefetch=2,
            grid=(B, E // G),
            in_specs=[pl.BlockSpec((1, S, G * HP), lambda b, e, a_r, ap_r: (b, 0, e))],
            out_specs=[pl.BlockSpec((1, M, G * HP), lambda b, e, a_r, ap_r: (b, 0, e)),
                       pl.BlockSpec((1, M, G * HP), lambda b, e, a_r, ap_r: (b, 0, e))],
        ),
        out_shape=[jax.ShapeDtypeStruct((B, M, EH), F32),
                   jax.ShapeDtypeStruct((B, M, EH), F32)],
    )(a_flat, ap_flat, v)

    MT = 4
    oe, oo = pl.pallas_call(
        _final_body,
        grid=(B, MT),
        in_specs=[pl.BlockSpec((1, M // MT, EH), lambda b, t: (b, t, 0)),
                  pl.BlockSpec((1, M // MT, EH), lambda b, t: (b, t, 0)),
                  pl.BlockSpec((1, 1, EH), lambda b, t: (b, 0, 0)),
                  pl.BlockSpec((EH, D), lambda b, t: (0, 0)),
                  pl.BlockSpec((1, D), lambda b, t: (0, 0))],
        out_specs=[pl.BlockSpec((1, M // MT, D), lambda b, t: (b, t, 0)),
                   pl.BlockSpec((1, M // MT, D), lambda b, t: (b, t, 0))],
        out_shape=[jax.ShapeDtypeStruct((B, M, D), F32),
                   jax.ShapeDtypeStruct((B, M, D), F32)],
    )(ge, go, w_row, wl2, bl3)

    return jnp.stack([oe, oo], axis=2).reshape(B, S, D)
